# baseline (speedup 1.0000x reference)
import jax
import jax.numpy as jnp
from jax.experimental import pallas as pl

NUM_CLASSES_ = 1000
ROW_BLOCK = 256


def _onehot_block(labels_ref, out_ref):
    labels = labels_ref[...]
    iota = jax.lax.broadcasted_iota(jnp.int32, (1, 1, NUM_CLASSES_), 2)
    out_ref[...] = (labels[:, :, None] == iota).astype(jnp.float32)


def kernel(labels):
    n, k = labels.shape
    grid = (n // ROW_BLOCK,)
    return pl.pallas_call(
        _onehot_block,
        grid=grid,
        in_specs=[pl.BlockSpec((ROW_BLOCK, k), lambda i: (i, 0))],
        out_specs=pl.BlockSpec((ROW_BLOCK, k, NUM_CLASSES_), lambda i: (i, 0, 0)),
        out_shape=jax.ShapeDtypeStruct((n, k, NUM_CLASSES_), jnp.float32),
    )(labels)


# speedup vs baseline: 1.0091x; 1.0091x over previous
import jax
import jax.numpy as jnp
from jax.experimental import pallas as pl

NUM_CLASSES_ = 1000
ROW_BLOCK = 64


def _onehot_block(labels_ref, out_ref):
    labels = labels_ref[...]
    iota = jax.lax.broadcasted_iota(jnp.int32, (1, 1, NUM_CLASSES_), 2)
    out_ref[...] = (labels[:, :, None] == iota).astype(jnp.float32)


def kernel(labels):
    n, k = labels.shape
    grid = (n // ROW_BLOCK,)
    return pl.pallas_call(
        _onehot_block,
        grid=grid,
        in_specs=[pl.BlockSpec((ROW_BLOCK, k), lambda i: (i, 0))],
        out_specs=pl.BlockSpec((ROW_BLOCK, k, NUM_CLASSES_), lambda i: (i, 0, 0)),
        out_shape=jax.ShapeDtypeStruct((n, k, NUM_CLASSES_), jnp.float32),
    )(labels)
